# Initial kernel scaffold; baseline (speedup 1.0000x reference)
import jax
import jax.numpy as jnp
from jax.experimental import pallas as pl


def kernel(x, edge_index, edge_attr, batch, poles_list, zeros_list, params):
    raise NotImplementedError("write your pallas kernel here")


# speedup vs baseline: 1.6577x; 1.6577x over previous
import functools

import jax
import jax.numpy as jnp
from jax import lax
from jax.experimental import pallas as pl
from jax.experimental.pallas import tpu as pltpu
from jax.experimental.pallas import tpu_sc as plsc

F32 = jnp.float32
_HIGH = lax.Precision.HIGHEST


def _dot(a, b, dims=None, prec=None):
    dn = (((1,), (0,)), ((), ())) if dims is None else dims
    return lax.dot_general(a, b, dn, precision=prec, preferred_element_type=F32)


def _k0_call(batch_col, x, w0h):
    n, nf = x.shape
    h = w0h.shape[1]
    blk = 1000
    grid = (n // blk,)

    def body(b_ref, x_ref, w_ref, cnt_ref, g0_ref):
        i = pl.program_id(0)

        @pl.when(i == 0)
        def _():
            cnt_ref[...] = jnp.zeros_like(cnt_ref)

        bcol = b_ref[0]
        gi = lax.broadcasted_iota(jnp.int32, (1, 64), 1)
        oh = (bcol == gi).astype(F32)
        ones = jnp.ones((blk, 1), F32)
        cnt_ref[...] += _dot(oh, ones, (((0,), (0,)), ((), ())), prec=_HIGH)
        g0_ref[...] = _dot(x_ref[...], w_ref[...])

    return pl.pallas_call(
        body,
        grid=grid,
        in_specs=[
            pl.BlockSpec((1, blk, 1), lambda i: (i, 0, 0)),
            pl.BlockSpec((blk, nf), lambda i: (i, 0)),
            pl.BlockSpec((nf, h), lambda i: (0, 0)),
        ],
        out_specs=[
            pl.BlockSpec((64, 1), lambda i: (0, 0)),
            pl.BlockSpec((blk, h), lambda i: (i, 0)),
        ],
        out_shape=[
            jax.ShapeDtypeStruct((64, 1), F32),
            jax.ShapeDtypeStruct((n, h), F32),
        ],
    )(batch_col, x, w0h)


def _edge_call(ea, src3, rp_lo, rp_hi, we, bm):
    e, ef = ea.shape
    h = we.shape[1]
    blk = 1280
    grid = (e // blk,)

    def body(ea_ref, s_ref, lo_ref, hi_ref, w_ref, b_ref,
             a0_ref, a1_ref, a2_ref, hv_ref, ec_ref):
        i = pl.program_id(0)

        @pl.when(i == 0)
        def _():
            hv_ref[...] = jnp.zeros_like(hv_ref)
            ec_ref[...] = jnp.zeros_like(ec_ref)

        eab = ea_ref[...]
        for l, aref in enumerate((a0_ref, a1_ref, a2_ref)):
            aref[...] = _dot(eab, w_ref[l * ef:(l + 1) * ef, :]) + b_ref[l:l + 1, :]
        sv = s_ref[0]
        oh = ((sv >= lo_ref[...]) & (sv < hi_ref[...])).astype(F32)
        hv_ref[...] += _dot(oh, eab, prec=_HIGH)
        ec_ref[...] += jnp.sum(oh, axis=1, keepdims=True)

    return pl.pallas_call(
        body,
        grid=grid,
        in_specs=[
            pl.BlockSpec((blk, ef), lambda i: (i, 0)),
            pl.BlockSpec((1, 1, blk), lambda i: (i, 0, 0)),
            pl.BlockSpec((64, 1), lambda i: (0, 0)),
            pl.BlockSpec((64, 1), lambda i: (0, 0)),
            pl.BlockSpec((3 * ef, h), lambda i: (0, 0)),
            pl.BlockSpec((3, h), lambda i: (0, 0)),
        ],
        out_specs=[
            pl.BlockSpec((blk, h), lambda i: (i, 0)),
            pl.BlockSpec((blk, h), lambda i: (i, 0)),
            pl.BlockSpec((blk, h), lambda i: (i, 0)),
            pl.BlockSpec((64, ef), lambda i: (0, 0)),
            pl.BlockSpec((64, 1), lambda i: (0, 0)),
        ],
        out_shape=[
            jax.ShapeDtypeStruct((e, h), F32),
            jax.ShapeDtypeStruct((e, h), F32),
            jax.ShapeDtypeStruct((e, h), F32),
            jax.ShapeDtypeStruct((64, ef), F32),
            jax.ShapeDtypeStruct((64, 1), F32),
        ],
    )(ea, src3, rp_lo, rp_hi, we, bm)


def _update_call(hprev, agg, wut, wub, bu, wmnext):
    n, hin = hprev.shape
    h = wub.shape[1]
    blk = 1000
    grid = (n // blk,)

    hw = wmnext.shape[1]

    def body(h_ref, a_ref, wt_ref, wb_ref, b_ref, wm_ref, hn_ref, gn_ref):
        hn = _dot(h_ref[...], wt_ref[...]) + _dot(a_ref[...], wb_ref[...])
        hn = jnp.maximum(hn + b_ref[...], 0.0)
        hn_ref[...] = hn
        gn_ref[...] = _dot(hn, wm_ref[...])

    return pl.pallas_call(
        body,
        grid=grid,
        in_specs=[
            pl.BlockSpec((blk, hin), lambda i: (i, 0)),
            pl.BlockSpec((blk, h), lambda i: (i, 0)),
            pl.BlockSpec((hin, h), lambda i: (0, 0)),
            pl.BlockSpec((h, h), lambda i: (0, 0)),
            pl.BlockSpec((1, h), lambda i: (0, 0)),
            pl.BlockSpec((h, hw), lambda i: (0, 0)),
        ],
        out_specs=[
            pl.BlockSpec((blk, h), lambda i: (i, 0)),
            pl.BlockSpec((blk, hw), lambda i: (i, 0)),
        ],
        out_shape=[
            jax.ShapeDtypeStruct((n, h), F32),
            jax.ShapeDtypeStruct((n, hw), F32),
        ],
    )(hprev, agg, wut, wub, bu, wmnext)


def _update_pool_call(hprev, agg, wut, wub, bu, batch_col, batch2):
    n, h = hprev.shape
    blk = 1000
    grid = (n // blk,)

    def body(h_ref, a_ref, wt_ref, wb_ref, b_ref, bc_ref, bs_ref,
             sum_ref, max_ref):
        i = pl.program_id(0)

        @pl.when(i == 0)
        def _():
            sum_ref[...] = jnp.zeros_like(sum_ref)
            max_ref[...] = jnp.full_like(max_ref, -1e30)

        hn = _dot(h_ref[...], wt_ref[...]) + _dot(a_ref[...], wb_ref[...])
        hn = jnp.maximum(hn + b_ref[...], 0.0)
        bcol = bc_ref[0]
        gi = lax.broadcasted_iota(jnp.int32, (1, 64), 1)
        oh = (bcol == gi).astype(F32)
        sum_ref[...] += _dot(oh, hn, (((0,), (0,)), ((), ())), prec=_HIGH)
        g_lo = bs_ref[0, 0, 0]
        g_hi = bs_ref[0, 0, blk - 1]

        def gbody(g, _):
            sel = jnp.where(bcol == g, hn, -1e30)
            mx = jnp.max(sel, axis=0, keepdims=True)
            max_ref[pl.ds(g, 1), :] = jnp.maximum(max_ref[pl.ds(g, 1), :], mx)
            return 0

        lax.fori_loop(g_lo, g_hi + 1, gbody, 0)

    return pl.pallas_call(
        body,
        grid=grid,
        in_specs=[
            pl.BlockSpec((blk, h), lambda i: (i, 0)),
            pl.BlockSpec((blk, h), lambda i: (i, 0)),
            pl.BlockSpec((h, h), lambda i: (0, 0)),
            pl.BlockSpec((h, h), lambda i: (0, 0)),
            pl.BlockSpec((1, h), lambda i: (0, 0)),
            pl.BlockSpec((1, blk, 1), lambda i: (i, 0, 0)),
            pl.BlockSpec((1, 1, blk), lambda i: (i, 0, 0),
                         memory_space=pltpu.SMEM),
        ],
        out_specs=[
            pl.BlockSpec((64, h), lambda i: (0, 0)),
            pl.BlockSpec((64, h), lambda i: (0, 0)),
        ],
        out_shape=[
            jax.ShapeDtypeStruct((64, h), F32),
            jax.ShapeDtypeStruct((64, h), F32),
        ],
    )(hprev, agg, wut, wub, bu, batch_col, batch2)


def _heads_call(hsum, hmax, cnt, hvnum, ecnt, poles_t, zeros_t, wd):
    names = ['Wt1', 'bt1', 'Wt2', 'bt2', 'Wtmu', 'btmu', 'Wtlv', 'btlv',
             'Wv1', 'bv1', 'Wv2', 'bv2', 'Wvmu', 'bvmu', 'Wvlv', 'bvlv',
             'Wp1', 'bp1', 'Wp2', 'bp2', 'Wp3', 'bp3',
             'Wz1', 'bz1', 'Wz2', 'bz2', 'Wz3', 'bz3',
             'Wc', 'bc', 'Wpmu', 'bpmu', 'Wplv', 'bplv']
    ws = [wd[k] for k in names]

    def body(hs_ref, hm_ref, c_ref, hv_ref, ec_ref, p_ref, z_ref, *refs):
        w = {k: r[...] for k, r in zip(names, refs[:len(names)])}
        mu_ref, lv_ref = refs[len(names):]
        cnt_ = c_ref[...]
        mean = hs_ref[...] / jnp.maximum(cnt_, 1.0)
        maxp = jnp.where(cnt_ > 0, hm_ref[...], 0.0)
        ht = jnp.concatenate([mean, maxp], axis=1)
        ht = jnp.maximum(_dot(ht, w['Wt1']) + w['bt1'], 0.0)
        ht = jnp.maximum(_dot(ht, w['Wt2']) + w['bt2'], 0.0)
        mu_t = _dot(ht, w['Wtmu']) + w['btmu']
        lv_t = _dot(ht, w['Wtlv']) + w['btlv']
        hv = hv_ref[...] / jnp.maximum(ec_ref[...], 1.0)
        hv = jnp.maximum(_dot(hv, w['Wv1']) + w['bv1'], 0.0)
        hv = jnp.maximum(_dot(hv, w['Wv2']) + w['bv2'], 0.0)
        mu_v = _dot(hv, w['Wvmu']) + w['bvmu']
        lv_v = _dot(hv, w['Wvlv']) + w['bvlv']

        def deepset(x_ref, w1, b1, w2, b2, w3, b3):
            a = jnp.maximum(_dot(x_ref[...], w1) + b1, 0.0)
            a = jnp.maximum(_dot(a, w2) + b2, 0.0)
            s = a[0:64, :]
            for i in range(1, 8):
                s = s + a[i * 64:(i + 1) * 64, :]
            return _dot(s, w3) + b3

        php = deepset(p_ref, w['Wp1'], w['bp1'], w['Wp2'], w['bp2'],
                      w['Wp3'], w['bp3'])
        phz = deepset(z_ref, w['Wz1'], w['bz1'], w['Wz2'], w['bz2'],
                      w['Wz3'], w['bz3'])
        hpz = jnp.concatenate([php, phz], axis=1)
        hpz = jnp.maximum(_dot(hpz, w['Wc']) + w['bc'], 0.0)
        mu_pz = _dot(hpz, w['Wpmu']) + w['bpmu']
        lv_pz = _dot(hpz, w['Wplv']) + w['bplv']
        mu_ref[...] = jnp.concatenate([mu_t, mu_v, mu_pz], axis=1)
        lv_ref[...] = jnp.concatenate([lv_t, lv_v, lv_pz], axis=1)

    return pl.pallas_call(
        body,
        in_specs=[
            pl.BlockSpec(a.shape, lambda: (0, 0))
            for a in (hsum, hmax, cnt, hvnum, ecnt, poles_t, zeros_t, *ws)
        ],
        out_specs=[
            pl.BlockSpec((64, 8), lambda: (0, 0)),
            pl.BlockSpec((64, 8), lambda: (0, 0)),
        ],
        out_shape=[
            jax.ShapeDtypeStruct((64, 8), F32),
            jax.ShapeDtypeStruct((64, 8), F32),
        ],
    )(hsum, hmax, cnt, hvnum, ecnt, poles_t, zeros_t, *ws)


_ECH = 32
_NTILES = 16


_SPLIT = 25040
_TPAIRS = 12560
_ZP = _TPAIRS // _NTILES
_CPAIRS = 32
_DUMP = _SPLIT // 2


def _sc_layer(g_tab, a_mat, src1d, dst1d):
    n, gw = g_tab.shape
    h = a_mat.shape[1]
    e = src1d.shape[0]
    nch = e // _ECH
    half0, half1 = _SPLIT, n - _SPLIT
    hp0, hp1 = half0 // 2, half1 // 2
    mesh = plsc.VectorSubcoreMesh(core_axis_name="c", subcore_axis_name="s",
                                  num_cores=2, num_subcores=_NTILES)
    assert 2 * _TPAIRS >= _SPLIT + 64 and gw == 2 * h

    @functools.partial(
        pl.kernel, mesh=mesh,
        out_type=jax.ShapeDtypeStruct((n // 2, gw), F32),
        scratch_types=[
            pltpu.VMEM((_ECH, h), F32),
            pltpu.VMEM((_ECH, gw), F32),
            pltpu.VMEM((_ECH,), jnp.int32),
            pltpu.VMEM((_ECH,), jnp.int32),
            pltpu.VMEM((_ECH, gw), F32),
            pltpu.VMEM((1, _ECH), jnp.int32),
            pltpu.VMEM((1, _ECH), jnp.int32),
            pltpu.VMEM((_CPAIRS, gw), F32),
            pltpu.VMEM((1, _CPAIRS), jnp.int32),
            pltpu.VMEM_SHARED((_TPAIRS, gw), F32),
            pltpu.SemaphoreType.DMA,
            pltpu.SemaphoreType.DMA,
        ],
    )
    def k(g_hbm, a_hbm, s_hbm, d_hbm, out_hbm,
          abuf, mbuf, sbuf, dbuf, mbuf2, ibuf, ibuf2, cbuf, rbuf, table,
          sem_g, sem_a):
        c = lax.axis_index("c")
        s = lax.axis_index("s")
        base = c * _SPLIT
        basep = c * hp0
        halfn = jnp.where(c == 0, half0, half1)
        hp = jnp.where(c == 0, hp0, hp1)

        def fill_ramp(off):
            for j in range(_CPAIRS // 16):
                rbuf[0, pl.ds(j * 16, 16)] = (off + j * 16
                                              + lax.iota(jnp.int32, 16))

        def zrow(r, _):
            for q in range(gw // 16):
                cbuf[r, pl.ds(q * 16, 16)] = jnp.zeros((16,), F32)
            return 0

        lax.fori_loop(0, _CPAIRS, zrow, 0)

        def z2row(r, _):
            for q in range(h // 16):
                mbuf2[r, pl.ds(q * 16, 16)] = jnp.zeros((16,), F32)
            return 0

        lax.fori_loop(0, _ECH, z2row, 0)

        def init_strip(t, _):
            off = jnp.minimum(s * _ZP + t * _CPAIRS, s * _ZP + _ZP - _CPAIRS)
            fill_ramp(off)
            pltpu.sync_copy(cbuf, table.at[rbuf.at[0]])
            return 0

        lax.fori_loop(0, (_ZP + _CPAIRS - 1) // _CPAIRS, init_strip, 0)
        plsc.subcore_barrier()
        lo = (s * nch) // _NTILES
        hi = ((s + 1) * nch) // _NTILES

        def chunk(i, _):
            pltpu.sync_copy(s_hbm.at[pl.ds(i * _ECH, _ECH)], sbuf)
            pltpu.sync_copy(d_hbm.at[pl.ds(i * _ECH, _ECH)], dbuf)
            gd = pltpu.async_copy(g_hbm.at[sbuf], mbuf, sem_g)
            ad = pltpu.async_copy(a_hbm.at[pl.ds(i * _ECH, _ECH)], abuf, sem_a)
            for j in range(_ECH // 16):
                d = dbuf[pl.ds(j * 16, 16)]
                loc = d - base
                even = (loc & 1) == 0
                inr = (loc >= 0) & (loc < halfn)
                ii = lax.iota(jnp.int32, 16) + (j * 16)
                dummy_e = _DUMP + (ii & 15)
                dummy_o = _DUMP + 16 + (ii & 15)
                pair = jnp.where(inr, loc >> 1, dummy_e)
                ibuf[0, pl.ds(j * 16, 16)] = jnp.where(even, pair, dummy_e)
                pair2 = jnp.where(inr, loc >> 1, dummy_o)
                ibuf2[0, pl.ds(j * 16, 16)] = jnp.where(even, dummy_o, pair2)
            ad.wait()
            gd.wait()

            def row(r, _):
                for q in range(h // 16):
                    g = mbuf[r, pl.ds(q * 16, 16)]
                    a = abuf[r, pl.ds(q * 16, 16)]
                    m = jnp.maximum(a + g, 0.0)
                    mbuf[r, pl.ds(q * 16, 16)] = m
                    mbuf2[r, pl.ds(h + q * 16, 16)] = m
                return 0

            lax.fori_loop(0, _ECH, row, 0)
            pltpu.sync_copy(mbuf, table.at[ibuf.at[0]], add=True)
            pltpu.sync_copy(mbuf2, table.at[ibuf2.at[0]], add=True)
            return 0

        lax.fori_loop(lo, hi, chunk, 0)
        plsc.subcore_barrier()

        nstrips = (hp + _CPAIRS - 1) // _CPAIRS

        def out_strip(j, _):
            kk = s + j * _NTILES

            @pl.when(kk < nstrips)
            def _():
                off = jnp.minimum(kk * _CPAIRS, hp - _CPAIRS)
                fill_ramp(off)
                pltpu.sync_copy(table.at[rbuf.at[0]], cbuf)
                pltpu.sync_copy(cbuf, out_hbm.at[pl.ds(basep + off, _CPAIRS)])

            return 0

        lax.fori_loop(0, (hp0 // _CPAIRS + _NTILES) // _NTILES, out_strip, 0)

    return k(g_tab, a_mat, src1d, dst1d)


def kernel(x, edge_index, edge_attr, batch, poles_list, zeros_list, params):
    n, nf = x.shape
    e, ef = edge_attr.shape
    b, pp, _ = poles_list.shape
    p = params
    h = p['bm0'].shape[0]

    src, dst = edge_index[0], edge_index[1]
    src3 = src.reshape(e // 1280, 1, 1280)
    batch_col = batch.reshape(n // 1000, 1000, 1)
    batch2 = batch.reshape(n // 1000, 1, 1000)
    poles_t = poles_list.transpose(1, 0, 2).reshape(pp * b, 2)
    zeros_t = zeros_list.transpose(1, 0, 2).reshape(pp * b, 2)

    we = jnp.concatenate([p['Wm0'][nf:], p['Wm1'][h:], p['Wm2'][h:]], axis=0)
    bm = jnp.stack([p['bm0'], p['bm1'], p['bm2']])
    r1 = lambda v: v.reshape(1, -1)

    padg = lambda w: jnp.pad(w, ((0, 0), (0, 2 * h - w.shape[1])))
    cnt, g0 = _k0_call(batch_col, x, padg(p['Wm0'][:nf]))
    rp = jnp.concatenate([jnp.zeros((1,), jnp.int32),
                          jnp.cumsum(cnt[:, 0].astype(jnp.int32))])
    rp_lo = rp[:64].reshape(64, 1)
    rp_hi = rp[1:].reshape(64, 1)

    a0, a1, a2, hvnum, ecnt = _edge_call(edge_attr, src3, rp_lo, rp_hi, we, bm)

    agg0 = _sc_layer(g0, a0, src, dst).reshape(n, h)
    h1, g1 = _update_call(x, agg0, p['Wu0'][:nf], p['Wu0'][nf:],
                          r1(p['bu0']), padg(p['Wm1'][:h]))
    agg1 = _sc_layer(g1, a1, src, dst).reshape(n, h)
    h2, g2 = _update_call(h1, agg1, p['Wu1'][:h], p['Wu1'][h:],
                          r1(p['bu1']), padg(p['Wm2'][:h]))
    agg2 = _sc_layer(g2, a2, src, dst).reshape(n, h)
    hsum, hmax = _update_pool_call(h2, agg2, p['Wu2'][:h], p['Wu2'][h:],
                                   r1(p['bu2']), batch_col, batch2)

    wd = {k: (r1(v) if k.startswith('b') else v) for k, v in p.items()}
    mu, lv = _heads_call(hsum, hmax, cnt, hvnum, ecnt, poles_t, zeros_t, wd)
    return mu, mu, lv
